# baseline (speedup 1.0000x reference)
import functools

import numpy as np
import jax
import jax.numpy as jnp
from jax.experimental import pallas as pl
from jax.experimental.pallas import tpu as pltpu

_MASK_PROB = 0.65
_MASK_LENGTH = 10
_MASK_SELECTION = "static"
_MASK_OTHER = 0.0
_NO_MASK_OVERLAP = False
_MASK_MIN_SPACE = 1
_MASK_CHANNEL_PROB = 0.1
_MASK_CHANNEL_LENGTH = 64
_MASK_CHANNEL_SELECTION = "static"
_MASK_CHANNEL_OTHER = 0.0
_NO_MASK_CHANNEL_OVERLAP = False
_MASK_CHANNEL_MIN_SPACE = 1

_WROWS = 512
_NBUF = 6
_PREF = 4
_SKIP = 4
_LN = 128


def _mask_indices_np(shape, padding_mask, mask_prob, mask_length, mask_type, mask_other, min_masks=0, no_overlap=False, min_space=0):
    bsz, all_sz = shape
    mask = np.full((bsz, all_sz), False)
    all_num_mask = int(mask_prob * all_sz / float(mask_length) + np.random.rand())
    all_num_mask = max(min_masks, all_num_mask)
    mask_idcs = []
    for i in range(bsz):
        if padding_mask is not None:
            sz = all_sz - int(padding_mask[i].sum())
            num_mask = int(mask_prob * sz / float(mask_length) + np.random.rand())
            num_mask = max(min_masks, num_mask)
        else:
            sz = all_sz
            num_mask = all_num_mask
        if mask_type == "static":
            lengths = np.full(num_mask, mask_length)
        elif mask_type == "uniform":
            lengths = np.random.randint(mask_other, mask_length * 2 + 1, size=num_mask)
        elif mask_type == "normal":
            lengths = np.random.normal(mask_length, mask_other, size=num_mask)
            lengths = np.asarray([max(1, int(round(x))) for x in lengths])
        elif mask_type == "poisson":
            lengths = np.random.poisson(mask_length, size=num_mask)
            lengths = np.asarray([int(round(x)) for x in lengths])
        else:
            raise Exception("unknown mask selection " + mask_type)
        if sum(lengths) == 0:
            lengths[0] = min(mask_length, sz - 1)
        if no_overlap:
            mask_idc = []

            def arrange(s, e, length, keep_length):
                span_start = np.random.randint(s, e - length)
                mask_idc.extend(span_start + j for j in range(length))
                new_parts = []
                if span_start - s - min_space >= keep_length:
                    new_parts.append((s, span_start - min_space + 1))
                if e - span_start - length - min_space > keep_length:
                    new_parts.append((span_start + length + min_space, e))
                return new_parts

            parts = [(0, sz)]
            min_length = min(lengths)
            for length in sorted(lengths, reverse=True):
                lens = np.fromiter((e - s if e - s >= length + min_space else 0 for s, e in parts), np.int_)
                l_sum = np.sum(lens)
                if l_sum == 0:
                    break
                probs = lens / np.sum(lens)
                c = np.random.choice(len(parts), p=probs)
                s, e = parts.pop(c)
                parts.extend(arrange(s, e, length, min_length))
            mask_idc = np.asarray(mask_idc)
        else:
            min_len = min(lengths)
            if sz - min_len <= num_mask:
                min_len = sz - num_mask - 1
            mask_idc = np.random.choice(sz - min_len, num_mask, replace=False)
            mask_idc = np.asarray([mask_idc[j] + offset for j in range(len(mask_idc)) for offset in range(lengths[j])])
        mask_idcs.append(np.unique(mask_idc[mask_idc < sz]))
    min_len = min([len(m) for m in mask_idcs])
    for i, mask_idc in enumerate(mask_idcs):
        if len(mask_idc) > min_len:
            mask_idc = np.random.choice(mask_idc, min_len, replace=False)
        mask[i, mask_idc] = True
    return mask


@functools.lru_cache(maxsize=None)
def _host_masks(B, T, C):
    np.random.seed(0)
    pm = np.zeros((B, T), dtype=bool)
    mt = _mask_indices_np((B, T), pm, _MASK_PROB, _MASK_LENGTH, _MASK_SELECTION,
                          _MASK_OTHER, min_masks=2, no_overlap=_NO_MASK_OVERLAP,
                          min_space=_MASK_MIN_SPACE)
    mc = _mask_indices_np((B, C), None, _MASK_CHANNEL_PROB, _MASK_CHANNEL_LENGTH,
                          _MASK_CHANNEL_SELECTION, _MASK_CHANNEL_OTHER,
                          no_overlap=_NO_MASK_CHANNEL_OVERLAP,
                          min_space=_MASK_CHANNEL_MIN_SPACE)
    return mt, mc


@functools.lru_cache(maxsize=None)
def _window_plan(B, T, C):
    mt, _ = _host_masks(B, T, C)
    NWIN = (B * T) // _WROWS
    segs, fills = [], []
    for w in range(NWIN):
        r0 = w * _WROWS
        b = r0 // T
        m = mt.reshape(-1)[r0:r0 + _WROWS]
        runs = []
        i = 0
        while i < _WROWS:
            if m[i]:
                j = i
                while j < _WROWS and m[j]:
                    j += 1
                runs.append((i, j - i))
                i = j
            else:
                i += 1
        fills.append(runs)
        sg, pos = [], 0
        for (rs, ln) in runs:
            if ln >= _SKIP:
                if rs > pos:
                    sg.append((pos, rs - pos))
                pos = rs + ln
        if pos < _WROWS:
            sg.append((pos, _WROWS - pos))
        segs.append((b, sg))
    return segs, fills


def _rewrite_body(x_hbm, emb8_v, mult8_v, out_hbm, bufs, embw, multw,
                  gsems, osems, *, plan, B, T, C):
    segs, fills = plan
    NWIN = len(segs)
    SUB = C // _LN
    WSUB = _WROWS * SUB
    indesc = {}
    outdesc = {}

    def fire_in(w):
        k = w % _NBUF
        r0 = w * _WROWS
        ds = []
        for (off, ln) in segs[w][1]:
            d = pltpu.make_async_copy(
                x_hbm.at[pl.ds((r0 + off) * SUB, ln * SUB)],
                bufs[k].at[pl.ds(off * SUB, ln * SUB)],
                gsems[k])
            d.start()
            ds.append(d)
        indesc[w] = ds

    def build_images(b):
        multw[pl.ds(0, SUB), :] = mult8_v[pl.ds(b * SUB, SUB), :]
        embw[pl.ds(0, SUB), :] = (emb8_v[...] * mult8_v[pl.ds(b * SUB, SUB), :])
        n = SUB
        while n < WSUB:
            m = min(n, WSUB - n)
            multw[pl.ds(n, m), :] = multw[pl.ds(0, m), :]
            embw[pl.ds(n, m), :] = embw[pl.ds(0, m), :]
            n += m

    for w in range(min(_PREF, NWIN)):
        fire_in(w)

    cur_b = segs[0][0]
    build_images(cur_b)

    for w in range(NWIN):
        k = w % _NBUF
        nxt = w + _PREF
        if nxt < NWIN:
            if nxt >= _NBUF:
                outdesc[nxt - _NBUF].wait()
            fire_in(nxt)
        if segs[w][0] != cur_b:
            cur_b = segs[w][0]
            build_images(cur_b)
        for d in indesc[w]:
            d.wait()
        bufs[k][...] = bufs[k][...] * multw[...]
        for (rs, ln) in fills[w]:
            bufs[k][pl.ds(rs * SUB, ln * SUB), :] = embw[pl.ds(0, ln * SUB), :]
        d = pltpu.make_async_copy(bufs[k], out_hbm.at[pl.ds(w * WSUB, WSUB)],
                                  osems[k])
        d.start()
        outdesc[w] = d

    for w in range(max(NWIN - _NBUF, 0), NWIN):
        outdesc[w].wait()


def kernel(x, padding_mask, mask_embedding):
    B, T, C = x.shape
    SUB = C // _LN
    mt_np, mc_np = _host_masks(B, T, C)
    mask_indices = jnp.asarray(mt_np)
    plan = _window_plan(B, T, C)

    mult8 = jnp.asarray((~mc_np).astype(np.float32)).reshape(B * SUB, _LN)
    emb8 = mask_embedding.astype(jnp.float32).reshape(SUB, _LN)

    xs = x.reshape(B * T * SUB, _LN)
    WSUB = _WROWS * SUB

    def body(x_hbm, emb_v, mult_v, out_hbm, *rest):
        bufs = list(rest[:_NBUF])
        embw = rest[_NBUF]
        multw = rest[_NBUF + 1]
        gsems = list(rest[_NBUF + 2:2 * _NBUF + 2])
        osems = list(rest[2 * _NBUF + 2:3 * _NBUF + 2])
        _rewrite_body(x_hbm, emb_v, mult_v, out_hbm, bufs, embw, multw,
                      gsems, osems, plan=plan, B=B, T=T, C=C)

    outf = pl.pallas_call(
        body,
        in_specs=[
            pl.BlockSpec(memory_space=pltpu.MemorySpace.HBM),
            pl.BlockSpec(memory_space=pltpu.MemorySpace.VMEM),
            pl.BlockSpec(memory_space=pltpu.MemorySpace.VMEM),
        ],
        out_specs=pl.BlockSpec(memory_space=pltpu.MemorySpace.HBM),
        out_shape=jax.ShapeDtypeStruct((B * T * SUB, _LN), jnp.float32),
        scratch_shapes=(
            [pltpu.VMEM((WSUB, _LN), jnp.float32) for _ in range(_NBUF + 2)]
            + [pltpu.SemaphoreType.DMA for _ in range(2 * _NBUF)]
        ),
    )(xs, emb8, mult8)

    return (outf.reshape(B, T, C), mask_indices)


# speedup vs baseline: 4.3271x; 4.3271x over previous
import functools

import numpy as np
import jax
import jax.numpy as jnp
from jax.experimental import pallas as pl
from jax.experimental.pallas import tpu as pltpu

_MASK_PROB = 0.65
_MASK_LENGTH = 10
_MASK_SELECTION = "static"
_MASK_OTHER = 0.0
_NO_MASK_OVERLAP = False
_MASK_MIN_SPACE = 1
_MASK_CHANNEL_PROB = 0.1
_MASK_CHANNEL_LENGTH = 64
_MASK_CHANNEL_SELECTION = "static"
_MASK_CHANNEL_OTHER = 0.0
_NO_MASK_CHANNEL_OVERLAP = False
_MASK_CHANNEL_MIN_SPACE = 1

_WROWS = 1024
_NBUF = 6
_PREF = 4


def _mask_indices_np(shape, padding_mask, mask_prob, mask_length, mask_type, mask_other, min_masks=0, no_overlap=False, min_space=0):
    bsz, all_sz = shape
    mask = np.full((bsz, all_sz), False)
    all_num_mask = int(mask_prob * all_sz / float(mask_length) + np.random.rand())
    all_num_mask = max(min_masks, all_num_mask)
    mask_idcs = []
    for i in range(bsz):
        if padding_mask is not None:
            sz = all_sz - int(padding_mask[i].sum())
            num_mask = int(mask_prob * sz / float(mask_length) + np.random.rand())
            num_mask = max(min_masks, num_mask)
        else:
            sz = all_sz
            num_mask = all_num_mask
        if mask_type == "static":
            lengths = np.full(num_mask, mask_length)
        elif mask_type == "uniform":
            lengths = np.random.randint(mask_other, mask_length * 2 + 1, size=num_mask)
        elif mask_type == "normal":
            lengths = np.random.normal(mask_length, mask_other, size=num_mask)
            lengths = np.asarray([max(1, int(round(x))) for x in lengths])
        elif mask_type == "poisson":
            lengths = np.random.poisson(mask_length, size=num_mask)
            lengths = np.asarray([int(round(x)) for x in lengths])
        else:
            raise Exception("unknown mask selection " + mask_type)
        if sum(lengths) == 0:
            lengths[0] = min(mask_length, sz - 1)
        if no_overlap:
            mask_idc = []

            def arrange(s, e, length, keep_length):
                span_start = np.random.randint(s, e - length)
                mask_idc.extend(span_start + j for j in range(length))
                new_parts = []
                if span_start - s - min_space >= keep_length:
                    new_parts.append((s, span_start - min_space + 1))
                if e - span_start - length - min_space > keep_length:
                    new_parts.append((span_start + length + min_space, e))
                return new_parts

            parts = [(0, sz)]
            min_length = min(lengths)
            for length in sorted(lengths, reverse=True):
                lens = np.fromiter((e - s if e - s >= length + min_space else 0 for s, e in parts), np.int_)
                l_sum = np.sum(lens)
                if l_sum == 0:
                    break
                probs = lens / np.sum(lens)
                c = np.random.choice(len(parts), p=probs)
                s, e = parts.pop(c)
                parts.extend(arrange(s, e, length, min_length))
            mask_idc = np.asarray(mask_idc)
        else:
            min_len = min(lengths)
            if sz - min_len <= num_mask:
                min_len = sz - num_mask - 1
            mask_idc = np.random.choice(sz - min_len, num_mask, replace=False)
            mask_idc = np.asarray([mask_idc[j] + offset for j in range(len(mask_idc)) for offset in range(lengths[j])])
        mask_idcs.append(np.unique(mask_idc[mask_idc < sz]))
    min_len = min([len(m) for m in mask_idcs])
    for i, mask_idc in enumerate(mask_idcs):
        if len(mask_idc) > min_len:
            mask_idc = np.random.choice(mask_idc, min_len, replace=False)
        mask[i, mask_idc] = True
    return mask


@functools.lru_cache(maxsize=None)
def _host_masks(B, T, C):
    np.random.seed(0)
    pm = np.zeros((B, T), dtype=bool)
    mt = _mask_indices_np((B, T), pm, _MASK_PROB, _MASK_LENGTH, _MASK_SELECTION,
                          _MASK_OTHER, min_masks=2, no_overlap=_NO_MASK_OVERLAP,
                          min_space=_MASK_MIN_SPACE)
    mc = _mask_indices_np((B, C), None, _MASK_CHANNEL_PROB, _MASK_CHANNEL_LENGTH,
                          _MASK_CHANNEL_SELECTION, _MASK_CHANNEL_OTHER,
                          no_overlap=_NO_MASK_CHANNEL_OVERLAP,
                          min_space=_MASK_CHANNEL_MIN_SPACE)
    return mt, mc


@functools.lru_cache(maxsize=None)
def _window_plan(B, T, C):
    mt, mc = _host_masks(B, T, C)
    NWIN = (B * T) // _WROWS
    covers = []
    for b in range(B):
        idx = np.nonzero(mc[b])[0]
        if len(idx):
            c0 = (int(idx[0]) // 128) * 128
            c1 = -(-(int(idx[-1]) + 1) // 128) * 128
        else:
            c0, c1 = 0, 0
        covers.append((c0, c1 - c0))
    segs, fills = [], []
    for w in range(NWIN):
        r0 = w * _WROWS
        b = r0 // T
        m = mt.reshape(-1)[r0:r0 + _WROWS]
        runs = []
        i = 0
        while i < _WROWS:
            if m[i]:
                j = i
                while j < _WROWS and m[j]:
                    j += 1
                runs.append((i, j - i))
                i = j
            else:
                i += 1
        fills.append(runs)
        nblk = _WROWS // 8
        keep = [not m[i * 8:(i + 1) * 8].all() for i in range(nblk)]
        sg, i = [], 0
        while i < nblk:
            if keep[i]:
                j = i
                while j < nblk and keep[j]:
                    j += 1
                sg.append((i * 8, (j - i) * 8))
                i = j
            else:
                i += 1
        segs.append((b, sg))
    return segs, fills, covers


def _rewrite_body(x_hbm, emb_v, mult_v, out_hbm, bufs, gsems, osems, *,
                  plan, B, T, C):
    segs, fills, covers = plan
    NWIN = len(segs)
    indesc = {}
    outdesc = {}

    def fire_in(w):
        k = w % _NBUF
        r0 = w * _WROWS
        ds = []
        for (off, ln) in segs[w][1]:
            d = pltpu.make_async_copy(
                x_hbm.at[pl.ds(r0 + off, ln)],
                bufs[k].at[pl.ds(off, ln)],
                gsems[k])
            d.start()
            ds.append(d)
        indesc[w] = ds

    for w in range(min(_PREF, NWIN)):
        fire_in(w)

    for w in range(NWIN):
        k = w % _NBUF
        nxt = w + _PREF
        if nxt < NWIN:
            if nxt >= _NBUF:
                outdesc[nxt - _NBUF].wait()
            fire_in(nxt)
        for d in indesc[w]:
            d.wait()
        b = segs[w][0]
        mrow = emb_v[...] * mult_v[pl.ds(b, 1), :]
        c0, cw = covers[b]
        bufs[k][:, pl.ds(c0, cw)] = (bufs[k][:, pl.ds(c0, cw)]
                                     * mult_v[pl.ds(b, 1), pl.ds(c0, cw)])
        for (rs, ln) in fills[w]:
            bufs[k][pl.ds(rs, ln), :] = jnp.broadcast_to(mrow, (ln, C))
        d = pltpu.make_async_copy(bufs[k], out_hbm.at[pl.ds(w * _WROWS, _WROWS)],
                                  osems[k])
        d.start()
        outdesc[w] = d

    for w in range(max(NWIN - _NBUF, 0), NWIN):
        outdesc[w].wait()


def kernel(x, padding_mask, mask_embedding):
    B, T, C = x.shape
    mt_np, mc_np = _host_masks(B, T, C)
    mask_indices = jnp.asarray(mt_np)
    plan = _window_plan(B, T, C)

    mult = jnp.asarray((~mc_np).astype(np.float32))
    emb1 = mask_embedding.astype(jnp.float32).reshape(1, C)

    xf = x.reshape(B * T, C)

    def body(x_hbm, emb_v, mult_v, out_hbm, *rest):
        bufs = list(rest[:_NBUF])
        gsems = list(rest[_NBUF:2 * _NBUF])
        osems = list(rest[2 * _NBUF:3 * _NBUF])
        _rewrite_body(x_hbm, emb_v, mult_v, out_hbm, bufs, gsems, osems,
                      plan=plan, B=B, T=T, C=C)

    outf = pl.pallas_call(
        body,
        in_specs=[
            pl.BlockSpec(memory_space=pltpu.MemorySpace.HBM),
            pl.BlockSpec(memory_space=pltpu.MemorySpace.VMEM),
            pl.BlockSpec(memory_space=pltpu.MemorySpace.VMEM),
        ],
        out_specs=pl.BlockSpec(memory_space=pltpu.MemorySpace.HBM),
        out_shape=jax.ShapeDtypeStruct((B * T, C), jnp.float32),
        scratch_shapes=(
            [pltpu.VMEM((_WROWS, C), jnp.float32) for _ in range(_NBUF)]
            + [pltpu.SemaphoreType.DMA for _ in range(2 * _NBUF)]
        ),
    )(xf, emb1, mult)

    return (outf.reshape(B, T, C), mask_indices)


# speedup vs baseline: 4.4677x; 1.0325x over previous
import functools

import numpy as np
import jax
import jax.numpy as jnp
from jax.experimental import pallas as pl
from jax.experimental.pallas import tpu as pltpu

_MASK_PROB = 0.65
_MASK_LENGTH = 10
_MASK_SELECTION = "static"
_MASK_OTHER = 0.0
_NO_MASK_OVERLAP = False
_MASK_MIN_SPACE = 1
_MASK_CHANNEL_PROB = 0.1
_MASK_CHANNEL_LENGTH = 64
_MASK_CHANNEL_SELECTION = "static"
_MASK_CHANNEL_OTHER = 0.0
_NO_MASK_CHANNEL_OVERLAP = False
_MASK_CHANNEL_MIN_SPACE = 1

_WROWS = 2048
_NBUF = 6
_PREF = 4


def _mask_indices_np(shape, padding_mask, mask_prob, mask_length, mask_type, mask_other, min_masks=0, no_overlap=False, min_space=0):
    bsz, all_sz = shape
    mask = np.full((bsz, all_sz), False)
    all_num_mask = int(mask_prob * all_sz / float(mask_length) + np.random.rand())
    all_num_mask = max(min_masks, all_num_mask)
    mask_idcs = []
    for i in range(bsz):
        if padding_mask is not None:
            sz = all_sz - int(padding_mask[i].sum())
            num_mask = int(mask_prob * sz / float(mask_length) + np.random.rand())
            num_mask = max(min_masks, num_mask)
        else:
            sz = all_sz
            num_mask = all_num_mask
        if mask_type == "static":
            lengths = np.full(num_mask, mask_length)
        elif mask_type == "uniform":
            lengths = np.random.randint(mask_other, mask_length * 2 + 1, size=num_mask)
        elif mask_type == "normal":
            lengths = np.random.normal(mask_length, mask_other, size=num_mask)
            lengths = np.asarray([max(1, int(round(x))) for x in lengths])
        elif mask_type == "poisson":
            lengths = np.random.poisson(mask_length, size=num_mask)
            lengths = np.asarray([int(round(x)) for x in lengths])
        else:
            raise Exception("unknown mask selection " + mask_type)
        if sum(lengths) == 0:
            lengths[0] = min(mask_length, sz - 1)
        if no_overlap:
            mask_idc = []

            def arrange(s, e, length, keep_length):
                span_start = np.random.randint(s, e - length)
                mask_idc.extend(span_start + j for j in range(length))
                new_parts = []
                if span_start - s - min_space >= keep_length:
                    new_parts.append((s, span_start - min_space + 1))
                if e - span_start - length - min_space > keep_length:
                    new_parts.append((span_start + length + min_space, e))
                return new_parts

            parts = [(0, sz)]
            min_length = min(lengths)
            for length in sorted(lengths, reverse=True):
                lens = np.fromiter((e - s if e - s >= length + min_space else 0 for s, e in parts), np.int_)
                l_sum = np.sum(lens)
                if l_sum == 0:
                    break
                probs = lens / np.sum(lens)
                c = np.random.choice(len(parts), p=probs)
                s, e = parts.pop(c)
                parts.extend(arrange(s, e, length, min_length))
            mask_idc = np.asarray(mask_idc)
        else:
            min_len = min(lengths)
            if sz - min_len <= num_mask:
                min_len = sz - num_mask - 1
            mask_idc = np.random.choice(sz - min_len, num_mask, replace=False)
            mask_idc = np.asarray([mask_idc[j] + offset for j in range(len(mask_idc)) for offset in range(lengths[j])])
        mask_idcs.append(np.unique(mask_idc[mask_idc < sz]))
    min_len = min([len(m) for m in mask_idcs])
    for i, mask_idc in enumerate(mask_idcs):
        if len(mask_idc) > min_len:
            mask_idc = np.random.choice(mask_idc, min_len, replace=False)
        mask[i, mask_idc] = True
    return mask


@functools.lru_cache(maxsize=None)
def _host_masks(B, T, C):
    np.random.seed(0)
    pm = np.zeros((B, T), dtype=bool)
    mt = _mask_indices_np((B, T), pm, _MASK_PROB, _MASK_LENGTH, _MASK_SELECTION,
                          _MASK_OTHER, min_masks=2, no_overlap=_NO_MASK_OVERLAP,
                          min_space=_MASK_MIN_SPACE)
    mc = _mask_indices_np((B, C), None, _MASK_CHANNEL_PROB, _MASK_CHANNEL_LENGTH,
                          _MASK_CHANNEL_SELECTION, _MASK_CHANNEL_OTHER,
                          no_overlap=_NO_MASK_CHANNEL_OVERLAP,
                          min_space=_MASK_CHANNEL_MIN_SPACE)
    return mt, mc


@functools.lru_cache(maxsize=None)
def _window_plan(B, T, C):
    mt, mc = _host_masks(B, T, C)
    NWIN = (B * T) // _WROWS
    covers = []
    for b in range(B):
        idx = np.nonzero(mc[b])[0]
        if len(idx):
            c0 = (int(idx[0]) // 128) * 128
            c1 = -(-(int(idx[-1]) + 1) // 128) * 128
        else:
            c0, c1 = 0, 0
        covers.append((c0, c1 - c0))
    segs, fills = [], []
    for w in range(NWIN):
        r0 = w * _WROWS
        b = r0 // T
        m = mt.reshape(-1)[r0:r0 + _WROWS]
        runs = []
        i = 0
        while i < _WROWS:
            if m[i]:
                j = i
                while j < _WROWS and m[j]:
                    j += 1
                runs.append((i, j - i))
                i = j
            else:
                i += 1
        fills.append(runs)
        nblk = _WROWS // 8
        keep = [not m[i * 8:(i + 1) * 8].all() for i in range(nblk)]
        sg, i = [], 0
        while i < nblk:
            if keep[i]:
                j = i
                while j < nblk and keep[j]:
                    j += 1
                sg.append((i * 8, (j - i) * 8))
                i = j
            else:
                i += 1
        segs.append((b, sg))
    return segs, fills, covers


def _rewrite_body(x_hbm, emb_v, mult_v, out_hbm, bufs, gsems, osems, *,
                  plan, B, T, C):
    segs, fills, covers = plan
    NWIN = len(segs)
    indesc = {}
    outdesc = {}

    def fire_in(w):
        k = w % _NBUF
        r0 = w * _WROWS
        ds = []
        for (off, ln) in segs[w][1]:
            d = pltpu.make_async_copy(
                x_hbm.at[pl.ds(r0 + off, ln)],
                bufs[k].at[pl.ds(off, ln)],
                gsems[k])
            d.start()
            ds.append(d)
        indesc[w] = ds

    for w in range(min(_PREF, NWIN)):
        fire_in(w)

    for w in range(NWIN):
        k = w % _NBUF
        nxt = w + _PREF
        if nxt < NWIN:
            if nxt >= _NBUF:
                outdesc[nxt - _NBUF].wait()
            fire_in(nxt)
        for d in indesc[w]:
            d.wait()
        b = segs[w][0]
        mrow = emb_v[...] * mult_v[pl.ds(b, 1), :]
        c0, cw = covers[b]
        bufs[k][:, pl.ds(c0, cw)] = (bufs[k][:, pl.ds(c0, cw)]
                                     * mult_v[pl.ds(b, 1), pl.ds(c0, cw)])
        for (rs, ln) in fills[w]:
            bufs[k][pl.ds(rs, ln), :] = jnp.broadcast_to(mrow, (ln, C))
        d = pltpu.make_async_copy(bufs[k], out_hbm.at[pl.ds(w * _WROWS, _WROWS)],
                                  osems[k])
        d.start()
        outdesc[w] = d

    for w in range(max(NWIN - _NBUF, 0), NWIN):
        outdesc[w].wait()


def kernel(x, padding_mask, mask_embedding):
    B, T, C = x.shape
    mt_np, mc_np = _host_masks(B, T, C)
    mask_indices = jnp.asarray(mt_np)
    plan = _window_plan(B, T, C)

    mult = jnp.asarray((~mc_np).astype(np.float32))
    emb1 = mask_embedding.astype(jnp.float32).reshape(1, C)

    xf = x.reshape(B * T, C)

    def body(x_hbm, emb_v, mult_v, out_hbm, *rest):
        bufs = list(rest[:_NBUF])
        gsems = list(rest[_NBUF:2 * _NBUF])
        osems = list(rest[2 * _NBUF:3 * _NBUF])
        _rewrite_body(x_hbm, emb_v, mult_v, out_hbm, bufs, gsems, osems,
                      plan=plan, B=B, T=T, C=C)

    outf = pl.pallas_call(
        body,
        in_specs=[
            pl.BlockSpec(memory_space=pltpu.MemorySpace.HBM),
            pl.BlockSpec(memory_space=pltpu.MemorySpace.VMEM),
            pl.BlockSpec(memory_space=pltpu.MemorySpace.VMEM),
        ],
        out_specs=pl.BlockSpec(memory_space=pltpu.MemorySpace.HBM),
        out_shape=jax.ShapeDtypeStruct((B * T, C), jnp.float32),
        scratch_shapes=(
            [pltpu.VMEM((_WROWS, C), jnp.float32) for _ in range(_NBUF)]
            + [pltpu.SemaphoreType.DMA for _ in range(2 * _NBUF)]
        ),
    )(xf, emb1, mult)

    return (outf.reshape(B, T, C), mask_indices)
